# baseline (speedup 1.0000x reference)
import functools

import jax
import jax.numpy as jnp
from jax import lax
from jax.experimental import pallas as pl
from jax.experimental.pallas import tpu as pltpu
from jax.experimental.pallas import tpu_sc as plsc

NC = 2
NS = 16
L = 16
NW = NC * NS
B = 4096
BPW = B // NW
D = 16
N_ITEMS = 100000
MEAN_BLK = 51200


def _sigmoid(x):
    return 1.0 / (1.0 + jnp.exp(-x))


def _mean_body(dooft_ref, out_ref):
    out_ref[...] = jnp.sum(dooft_ref[...], axis=0) * (1.0 / D)


_doof_mean = pl.pallas_call(
    _mean_body,
    grid=((N_ITEMS + MEAN_BLK - 1) // MEAN_BLK,),
    in_specs=[pl.BlockSpec((D, MEAN_BLK), lambda i: (0, i))],
    out_specs=pl.BlockSpec((MEAN_BLK,), lambda i: (i,)),
    out_shape=jax.ShapeDtypeStruct((N_ITEMS,), jnp.float32),
)


def _user_body(user_hbm, item_hbm, ts_hbm, cs_hbm, drs_hbm,
               st_hbm, fw_hbm,
               uidx, iidx, ts_v, cs_v, drs_v, st_v, fw_v, sem):
    wid = lax.axis_index("s") * NC + lax.axis_index("c")
    base = wid * BPW

    pltpu.sync_copy(user_hbm.at[pl.ds(base, BPW)], uidx)
    pltpu.sync_copy(item_hbm.at[pl.ds(base, BPW)], iidx)

    copies = [
        pltpu.async_copy(ts_hbm.at[uidx], ts_v, sem),
        pltpu.async_copy(cs_hbm.at[uidx], cs_v, sem),
        pltpu.async_copy(drs_hbm.at[iidx], drs_v, sem),
    ]
    for c in copies:
        c.wait()

    for j in range(BPW // L):
        sl = pl.ds(j * L, L)
        st_v[sl] = _sigmoid(ts_v[sl]) * _sigmoid(drs_v[sl])
        fw_v[sl] = _sigmoid(cs_v[sl])

    pltpu.sync_copy(st_v, st_hbm.at[pl.ds(base, BPW)])
    pltpu.sync_copy(fw_v, fw_hbm.at[pl.ds(base, BPW)])


_user_call = functools.partial(
    pl.kernel,
    out_type=(jax.ShapeDtypeStruct((B,), jnp.float32),
              jax.ShapeDtypeStruct((B,), jnp.float32)),
    mesh=plsc.VectorSubcoreMesh(
        core_axis_name="c", subcore_axis_name="s",
        num_cores=NC, num_subcores=NS),
    scratch_types=[
        pltpu.VMEM((BPW,), jnp.int32),
        pltpu.VMEM((BPW,), jnp.int32),
        pltpu.VMEM((BPW,), jnp.float32),
        pltpu.VMEM((BPW,), jnp.float32),
        pltpu.VMEM((BPW,), jnp.float32),
        pltpu.VMEM((BPW,), jnp.float32),
        pltpu.VMEM((BPW,), jnp.float32),
        pltpu.SemaphoreType.DMA,
    ],
)(_user_body)


def _final_body(item_hbm, dm_hbm, st_hbm, fw_hbm, out_hbm,
                iidx, dm_v, st_v, fw_v, out_v, sem):
    wid = lax.axis_index("s") * NC + lax.axis_index("c")
    base = wid * BPW

    pltpu.sync_copy(item_hbm.at[pl.ds(base, BPW)], iidx)
    c1 = pltpu.async_copy(dm_hbm.at[iidx], dm_v, sem)
    pltpu.sync_copy(st_hbm.at[pl.ds(base, BPW)], st_v)
    pltpu.sync_copy(fw_hbm.at[pl.ds(base, BPW)], fw_v)
    c1.wait()

    for j in range(BPW // L):
        sl = pl.ds(j * L, L)
        out_v[sl] = st_v[sl] + fw_v[sl] * dm_v[sl]

    pltpu.sync_copy(out_v, out_hbm.at[pl.ds(base, BPW)])


_final_call = functools.partial(
    pl.kernel,
    out_type=jax.ShapeDtypeStruct((B,), jnp.float32),
    mesh=plsc.VectorSubcoreMesh(
        core_axis_name="c", subcore_axis_name="s",
        num_cores=NC, num_subcores=NS),
    scratch_types=[
        pltpu.VMEM((BPW,), jnp.int32),
        pltpu.VMEM((BPW,), jnp.float32),
        pltpu.VMEM((BPW,), jnp.float32),
        pltpu.VMEM((BPW,), jnp.float32),
        pltpu.VMEM((BPW,), jnp.float32),
        pltpu.SemaphoreType.DMA,
    ],
)(_final_body)


def kernel(user, item, user_ts, user_cs, item_DRS, item_DOOF):
    doof_mean = _doof_mean(item_DOOF.T)
    score_time, fame_w = _user_call(user, item, user_ts, user_cs, item_DRS)
    return _final_call(item, doof_mean, score_time, fame_w)


# speedup vs baseline: 1.1883x; 1.1883x over previous
import functools

import jax
import jax.numpy as jnp
from jax import lax
from jax.experimental import pallas as pl
from jax.experimental.pallas import tpu as pltpu
from jax.experimental.pallas import tpu_sc as plsc

NC = 2
NS = 16
L = 16
NW = NC * NS
B = 4096
BPW = B // NW
D = 16
N_ITEMS = 100000
MEAN_BLK = 51200


def _sigmoid(x):
    return 1.0 / (1.0 + jnp.exp(-x))


def _mean_body(dooft_ref, out_ref):
    out_ref[...] = jnp.sum(dooft_ref[...], axis=0) * (1.0 / D)


_doof_mean = pl.pallas_call(
    _mean_body,
    grid=((N_ITEMS + MEAN_BLK - 1) // MEAN_BLK,),
    in_specs=[pl.BlockSpec((D, MEAN_BLK), lambda i: (0, i))],
    out_specs=pl.BlockSpec((MEAN_BLK,), lambda i: (i,)),
    out_shape=jax.ShapeDtypeStruct((N_ITEMS,), jnp.float32),
)


def _sc_body(user_hbm, item_hbm, ts_hbm, cs_hbm, drs_hbm, dm_hbm, out_hbm,
             uidx, iidx, ts_v, cs_v, drs_v, dm_v, out_v, sem):
    wid = lax.axis_index("s") * NC + lax.axis_index("c")
    base = wid * BPW

    pltpu.sync_copy(user_hbm.at[pl.ds(base, BPW)], uidx)
    pltpu.sync_copy(item_hbm.at[pl.ds(base, BPW)], iidx)

    copies = [
        pltpu.async_copy(ts_hbm.at[uidx], ts_v, sem),
        pltpu.async_copy(cs_hbm.at[uidx], cs_v, sem),
        pltpu.async_copy(drs_hbm.at[iidx], drs_v, sem),
        pltpu.async_copy(dm_hbm.at[iidx], dm_v, sem),
    ]
    for c in copies:
        c.wait()

    for j in range(BPW // L):
        sl = pl.ds(j * L, L)
        out_v[sl] = (
            _sigmoid(ts_v[sl]) * _sigmoid(drs_v[sl])
            + _sigmoid(cs_v[sl]) * dm_v[sl])

    pltpu.sync_copy(out_v, out_hbm.at[pl.ds(base, BPW)])


_sc_call = functools.partial(
    pl.kernel,
    out_type=jax.ShapeDtypeStruct((B,), jnp.float32),
    mesh=plsc.VectorSubcoreMesh(
        core_axis_name="c", subcore_axis_name="s",
        num_cores=NC, num_subcores=NS),
    scratch_types=[
        pltpu.VMEM((BPW,), jnp.int32),
        pltpu.VMEM((BPW,), jnp.int32),
        pltpu.VMEM((BPW,), jnp.float32),
        pltpu.VMEM((BPW,), jnp.float32),
        pltpu.VMEM((BPW,), jnp.float32),
        pltpu.VMEM((BPW,), jnp.float32),
        pltpu.VMEM((BPW,), jnp.float32),
        pltpu.SemaphoreType.DMA,
    ],
)(_sc_body)


def kernel(user, item, user_ts, user_cs, item_DRS, item_DOOF):
    doof_mean = _doof_mean(item_DOOF.T)
    return _sc_call(user, item, user_ts, user_cs, item_DRS, doof_mean)


# speedup vs baseline: 1.2116x; 1.0196x over previous
import functools

import jax
import jax.numpy as jnp
from jax import lax
from jax.experimental import pallas as pl
from jax.experimental.pallas import tpu as pltpu
from jax.experimental.pallas import tpu_sc as plsc

NC = 2
NS = 16
L = 16
NW = NC * NS
B = 4096
BPW = B // NW
D = 16
N_ITEMS = 100000
MEAN_BLK = 51200


def _sigmoid(x):
    return 1.0 / (1.0 + jnp.exp(-x))


def _mean_body(dooft_ref, out_ref):
    out_ref[...] = jnp.sum(dooft_ref[...], axis=0) * (1.0 / D)


_doof_mean = pl.pallas_call(
    _mean_body,
    grid=((N_ITEMS + MEAN_BLK - 1) // MEAN_BLK,),
    in_specs=[pl.BlockSpec((D, MEAN_BLK), lambda i: (0, i))],
    out_specs=pl.BlockSpec((MEAN_BLK,), lambda i: (i,)),
    out_shape=jax.ShapeDtypeStruct((N_ITEMS,), jnp.float32),
)


def _sc_body(user_hbm, item_hbm, ts_hbm, cs_hbm, drs_hbm, dm_hbm, out_hbm,
             uidx, iidx, ts_v, cs_v, drs_v, dm_v, out_v, sem):
    wid = lax.axis_index("s") * NC + lax.axis_index("c")
    base = wid * BPW

    i0 = pltpu.async_copy(user_hbm.at[pl.ds(base, BPW)], uidx, sem)
    i1 = pltpu.async_copy(item_hbm.at[pl.ds(base, BPW)], iidx, sem)
    i0.wait()
    i1.wait()

    copies = [
        pltpu.async_copy(ts_hbm.at[uidx], ts_v, sem),
        pltpu.async_copy(cs_hbm.at[uidx], cs_v, sem),
        pltpu.async_copy(drs_hbm.at[iidx], drs_v, sem),
        pltpu.async_copy(dm_hbm.at[iidx], dm_v, sem),
    ]
    for c in copies:
        c.wait()

    for j in range(BPW // L):
        sl = pl.ds(j * L, L)
        out_v[sl] = (
            _sigmoid(ts_v[sl]) * _sigmoid(drs_v[sl])
            + _sigmoid(cs_v[sl]) * dm_v[sl])

    pltpu.sync_copy(out_v, out_hbm.at[pl.ds(base, BPW)])


_sc_call = functools.partial(
    pl.kernel,
    out_type=jax.ShapeDtypeStruct((B,), jnp.float32),
    mesh=plsc.VectorSubcoreMesh(
        core_axis_name="c", subcore_axis_name="s",
        num_cores=NC, num_subcores=NS),
    scratch_types=[
        pltpu.VMEM((BPW,), jnp.int32),
        pltpu.VMEM((BPW,), jnp.int32),
        pltpu.VMEM((BPW,), jnp.float32),
        pltpu.VMEM((BPW,), jnp.float32),
        pltpu.VMEM((BPW,), jnp.float32),
        pltpu.VMEM((BPW,), jnp.float32),
        pltpu.VMEM((BPW,), jnp.float32),
        pltpu.SemaphoreType.DMA,
    ],
)(_sc_body)


def kernel(user, item, user_ts, user_cs, item_DRS, item_DOOF):
    doof_mean = _doof_mean(item_DOOF.T)
    return _sc_call(user, item, user_ts, user_cs, item_DRS, doof_mean)
